# baseline (speedup 1.0000x reference)
import functools

import jax
import jax.numpy as jnp
from jax import lax
from jax.experimental import pallas as pl
from jax.experimental.pallas import tpu as pltpu
from jax.experimental.pallas import tpu_sc as plsc

SVC = 10000
INST = 30000
NODE = 10000
D = 128
H = 128
OUTD = 128

ESVC_P = 160256
EBIG_P = 480256
PADC = 256

NTILES = 32
SPM_ROWS = 10064
TRASH = 10000
WINI = 10000

@functools.cache
def _mesh():
    return plsc.VectorSubcoreMesh(
        core_axis_name="c", subcore_axis_name="s",
        num_cores=2, num_subcores=16)


def _deg_body(svc_s, svc_d, in_s, in_d, ni_s, ni_d, ii_s, ii_d,
              o0, o1, o2, o3, o4, o5, o6, o7, cnt, idxv):
    cid = lax.axis_index("c")
    sid = lax.axis_index("s")
    wid = sid * 2 + cid
    ones = jnp.ones((16,), jnp.float32)
    zeros16 = jnp.zeros((16,), jnp.float32)
    specs = (
        (svc_s, o0, SVC, ESVC_P), (svc_d, o1, SVC, ESVC_P),
        (in_s, o2, INST, EBIG_P), (in_d, o3, NODE, EBIG_P),
        (ni_s, o4, NODE, EBIG_P), (ni_d, o5, INST, EBIG_P),
        (ii_s, o6, INST, EBIG_P), (ii_d, o7, INST, EBIG_P),
    )
    for iref, oref, space, epad in specs:
        nt = epad // NTILES

        def zb(i, _):
            cnt[pl.ds(i * 16, 16)] = zeros16
            return 0
        lax.fori_loop(0, (space + 16) // 16, zb, 0)

        pltpu.sync_copy(iref.at[pl.ds(wid * nt, nt)], idxv.at[pl.ds(0, nt)])

        def hincr(v):
            c, m = plsc.scan_count(v)
            plsc.addupdate_scatter(cnt, [v], c.astype(jnp.float32), mask=m)

        def hb(i, _):
            hincr(idxv[pl.ds(i * 32, 16)])
            hincr(idxv[pl.ds(i * 32 + 16, 16)])
            return 0
        lax.fori_loop(0, nt // 32, hb, 0)
        if nt % 32:
            hincr(idxv[pl.ds(nt - 16, 16)])

        pltpu.sync_copy(cnt.at[pl.ds(0, space)], oref.at[wid])


def _deg_kernel(svc_s, svc_d, in_s, in_d, ni_s, ni_d, ii_s, ii_d):
    spaces = (SVC, SVC, INST, NODE, NODE, INST, INST, INST)
    return pl.kernel(
        _deg_body,
        out_type=[jax.ShapeDtypeStruct((NTILES, s), jnp.float32)
                  for s in spaces],
        mesh=_mesh(),
        compiler_params=pltpu.CompilerParams(needs_layout_passes=False, use_tc_tiling_on_sc=False),
        scratch_types=[
            pltpu.VMEM((INST + 16,), jnp.float32),
            pltpu.VMEM((EBIG_P // NTILES,), jnp.int32),
        ],
    )(svc_s, svc_d, in_s, in_d, ni_s, ni_d, ii_s, ii_d)


def _mm_scale(x, w, degp, n_rows):
    br = 1024
    g = (n_rows + br - 1) // br

    def body(x_ref, w_ref, d_ref, o_ref):
        i = pl.program_id(0)
        s = jnp.sum(d_ref[...], axis=1, keepdims=True)
        gr2 = i * br + lax.broadcasted_iota(jnp.int32, (br, 1), 0)
        s = s - jnp.where(gr2 == 0, jnp.float32(PADC), jnp.float32(0.0))
        n = lax.rsqrt(jnp.maximum(s, 1.0))
        o_ref[...] = jnp.dot(x_ref[...] * n, w_ref[...],
                             preferred_element_type=jnp.float32)

    return pl.pallas_call(
        body,
        grid=(g,),
        in_specs=[
            pl.BlockSpec((br, D), lambda i: (i, 0)),
            pl.BlockSpec((D, H), lambda i: (0, 0)),
            pl.BlockSpec((br, NTILES), lambda i: (i, 0)),
        ],
        out_specs=pl.BlockSpec((br, H), lambda i: (i, 0)),
        out_shape=jax.ShapeDtypeStruct((n_rows, H), jnp.float32),
    )(x, w, degp)


def _agg_body(zeros_hbm, f_svc, f_in, f_ni, f_ii,
              svc_s, svc_d, in_s, in_d, ni_s, ni_d, ii_s, ii_d,
              o_svc, o_node, o_ni, o_ii,
              spm, srcv, dstv, srcc0, dstc0, srcc1, dstc1,
              rows0, rows1, sem0, sem1):
    cid = lax.axis_index("c")
    sid = lax.axis_index("s")
    wid = sid * 2 + cid
    iot = lax.iota(jnp.int32, 16)

    def zero_acc():
        pltpu.sync_copy(zeros_hbm.at[pl.ds(0, 64)], rows0)
        zbase = sid * (SPM_ROWS // 16)
        for j in range(9):
            pltpu.sync_copy(rows0, spm.at[pl.ds(zbase + j * 64, 64)])
        pltpu.sync_copy(rows0.at[pl.ds(0, 53)],
                        spm.at[pl.ds(zbase + 576, 53)])
        plsc.subcore_barrier()

    def process_chunks(feat, m_chunks, lim):
        def prep(sc_ref, dc_ref, base):
            for q in range(4):
                sc_ref[pl.ds(q * 16, 16)] = srcv[pl.ds(base + q * 16, 16)]
                dc_ref[pl.ds(q * 16, 16)] = dstv[pl.ds(base + q * 16, 16)]

        def wait_scatter(sc_ref, rows_ref, sem_ref, dc_ref):
            pltpu.make_async_copy(feat.at[sc_ref], rows_ref, sem_ref).wait()
            pltpu.sync_copy(rows_ref, spm.at[dc_ref], add=True)

        prep(srcc0, dstc0, jnp.minimum(0, lim))
        pltpu.async_copy(feat.at[srcc0], rows0, sem0)

        def body(kk, _):
            b = kk * 128
            prep(srcc1, dstc1, jnp.minimum(b + 64, lim))
            pltpu.async_copy(feat.at[srcc1], rows1, sem1)
            wait_scatter(srcc0, rows0, sem0, dstc0)
            prep(srcc0, dstc0, jnp.minimum(b + 128, lim))
            pltpu.async_copy(feat.at[srcc0], rows0, sem0)
            wait_scatter(srcc1, rows1, sem1, dstc1)
            return 0
        lax.fori_loop(0, m_chunks // 2, body, 0)
        wait_scatter(srcc0, rows0, sem0, dstc0)
        plsc.subcore_barrier()

    def pad_trash(at):
        for q in range(8):
            srcv[pl.ds(at + q * 16, 16)] = iot
            dstv[pl.ds(at + q * 16, 16)] = TRASH + ((iot + q * 16) & 63)

    def direct_phase(feat, src, dst, n_t, eoff, out, obase):
        zero_acc()
        pltpu.sync_copy(src.at[pl.ds(eoff, n_t)], srcv.at[pl.ds(0, n_t)])
        pltpu.sync_copy(dst.at[pl.ds(eoff, n_t)], dstv.at[pl.ds(0, n_t)])

        def tf(i, _):
            d_ = dstv[pl.ds(i * 16, 16)]
            tv = TRASH + ((iot + i * 16) & 63)
            dstv[pl.ds(i * 16, 16)] = jnp.where(d_ < SVC, d_, tv)
            return 0
        lax.fori_loop(0, n_t // 16, tf, 0)
        pad_trash(n_t)

        process_chunks(feat, (n_t + 63) // 64, n_t)
        pltpu.sync_copy(spm.at[pl.ds(sid * 625, 625)],
                        out.at[pl.ds(obase + sid * 625, 625)])
        plsc.subcore_barrier()

    direct_phase(f_svc, svc_s, svc_d, ESVC_P // 32, wid * (ESVC_P // 32),
                 o_svc, cid * SVC)
    direct_phase(f_in, in_s, in_d, EBIG_P // 32, wid * (EBIG_P // 32),
                 o_node, cid * NODE)

    n_t = EBIG_P // 32
    for src, dst, feat, out in ((ni_s, ni_d, f_ni, o_ni),
                                (ii_s, ii_d, f_ii, o_ii)):
        for w in range(3):
            pltpu.sync_copy(src.at[pl.ds(wid * n_t, n_t)],
                            srcv.at[pl.ds(0, n_t)])
            pltpu.sync_copy(dst.at[pl.ds(wid * n_t, n_t)],
                            dstv.at[pl.ds(0, n_t)])
            zero_acc()

            def scan_body(i, cnt):
                s_ = srcv[pl.ds(i * 16, 16)]
                d_ = dstv[pl.ds(i * 16, 16)]
                dw = d_ - w * WINI
                m = (dw >= 0) & (dw < WINI)
                plsc.store_compressed(srcv.at[pl.ds(cnt, 16)], s_, mask=m)
                plsc.store_compressed(dstv.at[pl.ds(cnt, 16)], dw, mask=m)
                return cnt + jnp.sum(m.astype(jnp.int32))
            cnt = lax.fori_loop(0, n_t // 16, scan_body, jnp.int32(0))
            pad_trash(cnt)

            process_chunks(feat, (cnt + 63) // 64, cnt)
            pltpu.sync_copy(
                spm.at[pl.ds(sid * 625, 625)],
                out.at[pl.ds(cid * INST + w * WINI + sid * 625, 625)])
            plsc.subcore_barrier()


def _agg_kernel(zeros_hbm, f_svc, f_in, f_ni, f_ii,
                svc_s, svc_d, in_s, in_d, ni_s, ni_d, ii_s, ii_d):
    return pl.kernel(
        _agg_body,
        out_type=[
            jax.ShapeDtypeStruct((2 * SVC, H), jnp.float32),
            jax.ShapeDtypeStruct((2 * NODE, H), jnp.float32),
            jax.ShapeDtypeStruct((2 * INST, H), jnp.float32),
            jax.ShapeDtypeStruct((2 * INST, H), jnp.float32),
        ],
        mesh=_mesh(),
        compiler_params=pltpu.CompilerParams(needs_layout_passes=False, use_tc_tiling_on_sc=False),
        scratch_types=[
            pltpu.VMEM_SHARED((SPM_ROWS, H), jnp.float32),
            pltpu.VMEM((EBIG_P // 32 + 128,), jnp.int32),
            pltpu.VMEM((EBIG_P // 32 + 128,), jnp.int32),
            pltpu.VMEM((64,), jnp.int32),
            pltpu.VMEM((64,), jnp.int32),
            pltpu.VMEM((64,), jnp.int32),
            pltpu.VMEM((64,), jnp.int32),
            pltpu.VMEM((64, H), jnp.float32),
            pltpu.VMEM((64, H), jnp.float32),
            pltpu.SemaphoreType.DMA,
            pltpu.SemaphoreType.DMA,
        ],
    )(zeros_hbm, f_svc, f_in, f_ni, f_ii,
      svc_s, svc_d, in_s, in_d, ni_s, ni_d, ii_s, ii_d)


def _leaky(h):
    return jnp.where(h >= 0, h, 0.01 * h)


def _proj_pair(wt, a0, a1, degp, b, acc, n_rows):
    br = 1024
    g = (n_rows + br - 1) // br

    def body(w_ref, a0_ref, a1_ref, d_ref, b_ref, acc_ref, o_ref):
        i = pl.program_id(0)
        s = jnp.sum(d_ref[...], axis=1, keepdims=True)
        n = lax.rsqrt(jnp.maximum(s, 1.0))
        h = (a0_ref[...] + a1_ref[...]) * n + b_ref[0:1, :]
        act = _leaky(h)
        gr2 = i * br + lax.broadcasted_iota(jnp.int32, (br, 1), 0)
        act = jnp.where(gr2 < n_rows, act, 0.0)
        gc2 = i * br + lax.broadcasted_iota(jnp.int32, (1, br), 1)
        w = jnp.where(gc2 < n_rows, w_ref[...], 0.0)

        @pl.when(i == 0)
        def _():
            o_ref[...] = acc_ref[...]
        o_ref[...] += jnp.dot(w, act, preferred_element_type=jnp.float32)

    return pl.pallas_call(
        body,
        grid=(g,),
        in_specs=[
            pl.BlockSpec((OUTD, br), lambda i: (0, i)),
            pl.BlockSpec((br, H), lambda i: (i, 0)),
            pl.BlockSpec((br, H), lambda i: (i, 0)),
            pl.BlockSpec((br, NTILES), lambda i: (i, 0)),
            pl.BlockSpec((8, H), lambda i: (0, 0)),
            pl.BlockSpec((OUTD, H), lambda i: (0, 0)),
        ],
        out_specs=pl.BlockSpec((OUTD, H), lambda i: (0, 0)),
        out_shape=jax.ShapeDtypeStruct((OUTD, H), jnp.float32),
    )(wt, a0, a1, degp, b, acc)


def _proj_two(wt, a1a, a1b, a2a, a2b, d1, d2, b1, b2, acc, n_rows):
    br = 1024
    g = (n_rows + br - 1) // br

    def body(w_ref, a1a_ref, a1b_ref, a2a_ref, a2b_ref,
             d1_ref, d2_ref, b1_ref, b2_ref, acc_ref, o_ref):
        i = pl.program_id(0)
        n1 = lax.rsqrt(jnp.maximum(
            jnp.sum(d1_ref[...], axis=1, keepdims=True), 1.0))
        n2 = lax.rsqrt(jnp.maximum(
            jnp.sum(d2_ref[...], axis=1, keepdims=True), 1.0))
        h = ((a1a_ref[...] + a1b_ref[...]) * n1 + b1_ref[0:1, :]
             + (a2a_ref[...] + a2b_ref[...]) * n2 + b2_ref[0:1, :])
        act = _leaky(h)
        gr2 = i * br + lax.broadcasted_iota(jnp.int32, (br, 1), 0)
        act = jnp.where(gr2 < n_rows, act, 0.0)
        gc2 = i * br + lax.broadcasted_iota(jnp.int32, (1, br), 1)
        w = jnp.where(gc2 < n_rows, w_ref[...], 0.0)

        @pl.when(i == 0)
        def _():
            o_ref[...] = acc_ref[...]
        o_ref[...] += jnp.dot(w, act, preferred_element_type=jnp.float32)

    return pl.pallas_call(
        body,
        grid=(g,),
        in_specs=[
            pl.BlockSpec((OUTD, br), lambda i: (0, i)),
            pl.BlockSpec((br, H), lambda i: (i, 0)),
            pl.BlockSpec((br, H), lambda i: (i, 0)),
            pl.BlockSpec((br, H), lambda i: (i, 0)),
            pl.BlockSpec((br, H), lambda i: (i, 0)),
            pl.BlockSpec((br, NTILES), lambda i: (i, 0)),
            pl.BlockSpec((br, NTILES), lambda i: (i, 0)),
            pl.BlockSpec((8, H), lambda i: (0, 0)),
            pl.BlockSpec((8, H), lambda i: (0, 0)),
            pl.BlockSpec((OUTD, H), lambda i: (0, 0)),
        ],
        out_specs=pl.BlockSpec((OUTD, H), lambda i: (0, 0)),
        out_shape=jax.ShapeDtypeStruct((OUTD, H), jnp.float32),
    )(wt, a1a, a1b, a2a, a2b, d1, d2, b1, b2, acc)


def _pad_edges(src, dst, dspace, epad):
    pe = epad - src.shape[0]
    sp = jnp.concatenate([src.astype(jnp.int32),
                          jnp.zeros((pe,), jnp.int32)])
    dp = jnp.concatenate([dst.astype(jnp.int32),
                          dspace + (jnp.arange(pe, dtype=jnp.int32) % 16)])
    return sp, dp


def kernel(svc_feat, instance_feat, node_feat,
           svc_call_src, svc_call_dst, inst_node_src, inst_node_dst,
           node_inst_src, node_inst_dst, inst_inst_src, inst_inst_dst,
           W_svc, b_svc, W_in, b_in, W_ni, b_ni, W_ii, b_ii,
           W_total, b_total):
    svc_s, svc_d = _pad_edges(svc_call_src, svc_call_dst, SVC, ESVC_P)
    in_s, in_d = _pad_edges(inst_node_src, inst_node_dst, NODE, EBIG_P)
    ni_s, ni_d = _pad_edges(node_inst_src, node_inst_dst, INST, EBIG_P)
    ii_s, ii_d = _pad_edges(inst_inst_src, inst_inst_dst, INST, EBIG_P)

    (d_svc_s, d_svc_d, d_in_s, d_in_d,
     d_ni_s, d_ni_d, d_ii_s, d_ii_d) = (
        d.T for d in _deg_kernel(
            svc_s, svc_d, in_s, in_d, ni_s, ni_d, ii_s, ii_d))

    f_svc = _mm_scale(svc_feat, W_svc, d_svc_s, SVC)
    f_in = _mm_scale(instance_feat, W_in, d_in_s, INST)
    f_ni = _mm_scale(node_feat, W_ni, d_ni_s, NODE)
    f_ii = _mm_scale(instance_feat, W_ii, d_ii_s, INST)

    zeros_hbm = jnp.zeros((128, H), jnp.float32)
    o_svc, o_node, o_ni, o_ii = _agg_kernel(
        zeros_hbm, f_svc, f_in, f_ni, f_ii,
        svc_s, svc_d, in_s, in_d, ni_s, ni_d, ii_s, ii_d)

    def bb(b):
        return jnp.broadcast_to(b[None, :], (8, H))

    bt = jnp.broadcast_to(b_total[:, None], (OUTD, H))
    p1 = _proj_pair(W_total[:, :SVC], o_svc[:SVC], o_svc[SVC:],
                    d_svc_d, bb(b_svc), bt, SVC)
    p2 = _proj_pair(W_total[:, SVC:SVC + NODE], o_node[:NODE], o_node[NODE:],
                    d_in_d, bb(b_in), p1, NODE)
    return _proj_two(W_total[:, SVC + NODE:], o_ni[:INST], o_ni[INST:],
                     o_ii[:INST], o_ii[INST:],
                     d_ni_d, d_ii_d, bb(b_ni), bb(b_ii), p2, INST)
